# Initial kernel scaffold; baseline (speedup 1.0000x reference)
import jax
import jax.numpy as jnp
from jax.experimental import pallas as pl


def kernel(x, patterns):
    raise NotImplementedError("write your pallas kernel here")


# speedup vs baseline: 1.1438x; 1.1438x over previous
import jax
import jax.numpy as jnp
from jax.experimental import pallas as pl
from jax.experimental.pallas import tpu as pltpu

_K = 100000
_D = 128
_B = 5000


def _body(x_ref, p_ref, out_ref, best_val, best_row):
    i = pl.program_id(0)

    @pl.when(i == 0)
    def _init():
        best_val[0] = -jnp.inf

    p = p_ref[...]
    x = x_ref[...]
    dots = jax.lax.dot_general(
        p, x, (((1,), (1,)), ((), ())),
        preferred_element_type=jnp.float32,
        precision=jax.lax.Precision.HIGHEST,
    )[:, 0]
    n2 = jnp.sum(p * p, axis=1)
    t = dots * jnp.abs(dots) / jnp.maximum(n2, 1e-16)

    local_max = jnp.max(t)
    better = local_max > best_val[0]

    @pl.when(better)
    def _upd():
        best_val[0] = local_max
        local_arg = jnp.argmax(t)
        onehot = (jax.lax.broadcasted_iota(jnp.int32, (_B, 1), 0)
                  == local_arg)
        best_row[...] = jnp.sum(
            jnp.where(onehot, p, 0.0), axis=0, keepdims=True)

    @pl.when(i == pl.num_programs(0) - 1)
    def _fin():
        out_ref[...] = best_row[...]


def kernel(x, patterns):
    out = pl.pallas_call(
        _body,
        grid=(_K // _B,),
        in_specs=[
            pl.BlockSpec((1, _D), lambda i: (0, 0)),
            pl.BlockSpec((_B, _D), lambda i: (i, 0)),
        ],
        out_specs=pl.BlockSpec((1, _D), lambda i: (0, 0)),
        out_shape=jax.ShapeDtypeStruct((1, _D), jnp.float32),
        scratch_shapes=[
            pltpu.SMEM((1,), jnp.float32),
            pltpu.VMEM((1, _D), jnp.float32),
        ],
        compiler_params=pltpu.CompilerParams(
            dimension_semantics=("arbitrary",),
        ),
    )(x.reshape(1, _D), patterns)
    return out[0]
